# baseline (speedup 1.0000x reference)
import jax
import jax.numpy as jnp
from jax import lax
from jax.experimental import pallas as pl
from jax.experimental.pallas import tpu as pltpu
from jax.experimental.pallas import tpu_sc as plsc

_N = 100000
_C = 5000
_D = 256
_B = 4096
_TEMP = 0.05
_MOM = 0.2

_NTILES = 32
_GCH = 128

_MCH = 128
_MFULL = _N // _MCH
_MTAIL = _N - _MFULL * _MCH
_MTAILS = _MFULL * _MCH
_MCPT = 25
_MTROWS = _MCPT * _MCH
_MBCH = _B // _MCH

_OWN = 160
_SCH = 1024
_LCHUNKS = 98
_LPAD = _LCHUNKS * _SCH
_TCHUNKS = _B // _SCH
_CAP = 2048


def _sc_mesh():
    return plsc.VectorSubcoreMesh(core_axis_name="c", subcore_axis_name="s")


def _mm_body(a_ref, b_ref, o_ref):
    o_ref[...] = lax.dot_general(
        a_ref[...], b_ref[...], (((1,), (1,)), ((), ())),
        preferred_element_type=jnp.float32) / _TEMP


def _logits(inputs, cluster_mean):
    bm, bn = 512, 512
    grid = (_B // bm, pl.cdiv(_C, bn))
    return pl.pallas_call(
        _mm_body,
        grid=grid,
        in_specs=[pl.BlockSpec((bm, _D), lambda i, j: (i, 0)),
                  pl.BlockSpec((bn, _D), lambda i, j: (j, 0))],
        out_specs=pl.BlockSpec((bm, bn), lambda i, j: (i, j)),
        out_shape=jax.ShapeDtypeStruct((_B, _C), jnp.float32),
        compiler_params=pltpu.CompilerParams(
            dimension_semantics=("parallel", "parallel")),
    )(inputs, cluster_mean)


def _gather_body(feat, idx2d, out, idx_v, rows_v, sem):
    wid = lax.axis_index("s") * 2 + lax.axis_index("c")
    pltpu.sync_copy(idx2d.at[wid], idx_v)
    pltpu.async_copy(feat.at[idx_v], rows_v, sem).wait()
    pltpu.sync_copy(rows_v, out.at[pl.ds(wid * _GCH, _GCH)])


def _gather_old(features, indexes2d):
    fn = pl.kernel(
        _gather_body,
        out_type=jax.ShapeDtypeStruct((_B, _D), jnp.float32),
        mesh=_sc_mesh(),
        scratch_types=[pltpu.VMEM((_GCH,), jnp.int32),
                       pltpu.VMEM((_GCH, _D), jnp.float32),
                       pltpu.SemaphoreType.DMA],
    )
    return fn(features, indexes2d)


def _upd_body(old_ref, inp_ref, msk_ref, new_ref, dlt_ref):
    old = old_ref[...]
    pre = _MOM * old + (1.0 - _MOM) * inp_ref[...]
    nrm = jnp.sqrt(jnp.sum(pre * pre, axis=1, keepdims=True))
    new = pre / jnp.maximum(nrm, 1e-12)
    new_ref[...] = new
    dlt_ref[...] = (new - old) * msk_ref[...]


def _update_rows(old, inputs, mask):
    bm = 512
    grid = (_B // bm,)
    out_sds = jax.ShapeDtypeStruct((_B, _D), jnp.float32)
    return pl.pallas_call(
        _upd_body,
        grid=grid,
        in_specs=[pl.BlockSpec((bm, _D), lambda i: (i, 0)),
                  pl.BlockSpec((bm, _D), lambda i: (i, 0)),
                  pl.BlockSpec((bm, 1), lambda i: (i, 0))],
        out_specs=[pl.BlockSpec((bm, _D), lambda i: (i, 0)),
                   pl.BlockSpec((bm, _D), lambda i: (i, 0))],
        out_shape=[out_sds, out_sds],
        compiler_params=pltpu.CompilerParams(dimension_semantics=("parallel",)),
    )(old, inputs, mask)


def _copy_body(feat, new_rows, rsrc, rdst, counts, feat_out,
               rows_v, rs_v, rd_v, cnt_v, sem):
    wid = lax.axis_index("s") * 2 + lax.axis_index("c")
    pltpu.sync_copy(counts.at[wid], cnt_v)

    n_my = jnp.where(wid == _NTILES - 1,
                     _MFULL - (_NTILES - 1) * _MCPT, _MCPT)

    def body1(c, carry):
        @pl.when(c < n_my)
        def _():
            gc = wid * _MCPT + c
            pltpu.sync_copy(feat.at[pl.ds(gc * _MCH, _MCH)], rows_v)
            pltpu.sync_copy(rows_v, feat_out.at[pl.ds(gc * _MCH, _MCH)])
        return carry

    lax.fori_loop(0, _MCPT, body1, 0)

    @pl.when(wid == _NTILES - 1)
    def _tail():
        pltpu.sync_copy(feat.at[pl.ds(_MTAILS, _MTAIL)],
                        rows_v.at[pl.ds(0, _MTAIL)])
        pltpu.sync_copy(rows_v.at[pl.ds(0, _MTAIL)],
                        feat_out.at[pl.ds(_MTAILS, _MTAIL)])

    n_upd = cnt_v[...][0]

    def body2(ch, carry):
        @pl.when(ch * _MCH < n_upd)
        def _():
            pltpu.sync_copy(rsrc.at[wid, ch], rs_v)
            pltpu.sync_copy(rdst.at[wid, ch], rd_v)
            pltpu.async_copy(new_rows.at[rs_v], rows_v, sem).wait()
            pltpu.sync_copy(rows_v, feat_out.at[rd_v])
        return carry

    lax.fori_loop(0, _MBCH, body2, 0)


def _copy_scatter(features, new_rows, rsrc, rdst, counts):
    fn = pl.kernel(
        _copy_body,
        out_type=jax.ShapeDtypeStruct((_N, _D), jnp.float32),
        mesh=_sc_mesh(),
        scratch_types=[pltpu.VMEM((_MCH, _D), jnp.float32),
                       pltpu.VMEM((_MCH,), jnp.int32),
                       pltpu.VMEM((_MCH,), jnp.int32),
                       pltpu.VMEM((16,), jnp.int32),
                       pltpu.SemaphoreType.DMA],
    )
    return fn(features, new_rows, rsrc, rdst, counts)


def _seg_body(feat, labels_p, targets_p, delta, zeros_d, zeros16, zeros_i,
              mean_out, flg_out,
              acc_v, cnt_v, flg_v, lbl_v, list_v, clist_v, rows_v, sem):
    wid = lax.axis_index("s") * 2 + lax.axis_index("c")
    lo = jnp.minimum(wid * _OWN, _C - _OWN)
    iota16 = lax.iota(jnp.int32, 16)
    ones16 = jnp.full((16,), 1.0, jnp.float32)

    pltpu.sync_copy(zeros_d, acc_v.at[pl.ds(0, 64)])
    pltpu.sync_copy(zeros_d, acc_v.at[pl.ds(64, 64)])
    pltpu.sync_copy(zeros_d.at[pl.ds(0, 32)], acc_v.at[pl.ds(128, 32)])
    for tbl in (cnt_v, flg_v):
        pltpu.sync_copy(zeros16, tbl.at[pl.ds(0, 64)])
        pltpu.sync_copy(zeros16, tbl.at[pl.ds(64, 64)])
        pltpu.sync_copy(zeros16.at[pl.ds(0, 32)], tbl.at[pl.ds(128, 32)])
    pltpu.sync_copy(zeros_i, list_v.at[pl.ds(0, _SCH)])
    pltpu.sync_copy(zeros_i, list_v.at[pl.ds(_SCH, _SCH)])

    def drain(src, cursor):
        def dbody(g, carry):
            @pl.when(g * 16 < cursor)
            def _():
                idxsl = list_v.at[pl.ds(g * 16, 16)]
                pltpu.async_copy(src.at[idxsl], rows_v, sem).wait()
                cvec = clist_v[pl.ds(g * 16, 16)]
                for k in range(16):
                    jk = cvec[k]

                    @pl.when(g * 16 + k < cursor)
                    def _(jk=jk, k=k):
                        for cc in range(16):
                            sl = pl.ds(cc * 16, 16)
                            acc_v[jk, sl] = acc_v[jk, sl] + rows_v[k, sl]
            return carry

        lax.fori_loop(0, _CAP // 16, dbody, 0)

    def compress_scan(tab, nch, src, match_table):
        def chunk_body(c, cursor):
            pltpu.sync_copy(tab.at[c], lbl_v)

            def grp(k, cur):
                tv = lbl_v[pl.ds(k * 16, 16)]
                rel = tv - lo
                m = (rel >= 0) & (rel < _OWN)
                relc = jnp.clip(rel, 0, _OWN - 1)
                plsc.addupdate_scatter(match_table, [relc, iota16], ones16,
                                       mask=m)
                rid = c * _SCH + k * 16 + iota16
                plsc.store_compressed(list_v.at[pl.ds(cur, 16)], rid, mask=m)
                plsc.store_compressed(clist_v.at[pl.ds(cur, 16)], relc, mask=m)
                return cur + plsc.all_reduce_population_count(m)[0]

            cursor = lax.fori_loop(0, _SCH // 16, grp, cursor)
            full = cursor > _CAP - _SCH

            @pl.when(full)
            def _():
                drain(src, cursor)

            return jnp.where(full, 0, cursor)

        cursor = lax.fori_loop(0, nch, chunk_body, 0)

        @pl.when(cursor > 0)
        def _():
            drain(src, cursor)

    compress_scan(targets_p, _TCHUNKS, delta, flg_v)
    compress_scan(labels_p, _LCHUNKS, feat, cnt_v)

    def fin(j, carry):
        cs = jnp.maximum(jnp.sum(cnt_v[j, pl.ds(0, 16)]), 1.0)
        for cc in range(16):
            sl = pl.ds(cc * 16, 16)
            acc_v[j, sl] = acc_v[j, sl] / cs
        return carry

    lax.fori_loop(0, _OWN, fin, 0)
    pltpu.sync_copy(acc_v.at[pl.ds(0, _OWN)], mean_out.at[pl.ds(lo, _OWN)])
    pltpu.sync_copy(flg_v.at[pl.ds(0, _OWN)], flg_out.at[pl.ds(lo, _OWN)])


def _seg(features, labels_p, targets_p, delta, zeros_d, zeros16, zeros_i):
    fn = pl.kernel(
        _seg_body,
        out_type=(jax.ShapeDtypeStruct((_C, _D), jnp.float32),
                  jax.ShapeDtypeStruct((_C, 16), jnp.float32)),
        mesh=_sc_mesh(),
        scratch_types=[pltpu.VMEM((160, _D), jnp.float32),
                       pltpu.VMEM((160, 16), jnp.float32),
                       pltpu.VMEM((160, 16), jnp.float32),
                       pltpu.VMEM((_SCH,), jnp.int32),
                       pltpu.VMEM((_CAP,), jnp.int32),
                       pltpu.VMEM((_CAP,), jnp.int32),
                       pltpu.VMEM((16, _D), jnp.float32),
                       pltpu.SemaphoreType.DMA],
        compiler_params=pltpu.CompilerParams(needs_layout_passes=False),
    )
    return fn(features, labels_p, targets_p, delta, zeros_d, zeros16, zeros_i)


def _merge_body(mean_ref, flg_ref, cm_ref, o_ref):
    f = jnp.sum(flg_ref[...], axis=1, keepdims=True)
    o_ref[...] = jnp.where(f > 0.5, mean_ref[...], cm_ref[...])


def _merge(mean, flg, cluster_mean):
    bn = 1000
    grid = (_C // bn,)
    wide = pl.BlockSpec((bn, _D), lambda i: (i, 0))
    narrow = pl.BlockSpec((bn, 16), lambda i: (i, 0))
    return pl.pallas_call(
        _merge_body,
        grid=grid,
        in_specs=[wide, narrow, wide],
        out_specs=wide,
        out_shape=jax.ShapeDtypeStruct((_C, _D), jnp.float32),
        compiler_params=pltpu.CompilerParams(dimension_semantics=("parallel",)),
    )(mean, flg, cluster_mean)


def kernel(inputs, indexes, IoU, targets, features, labels, cluster_mean):
    del IoU
    indexes = indexes.astype(jnp.int32)
    labels = labels.astype(jnp.int32)
    targets = targets.astype(jnp.int32)

    logits = _logits(inputs, cluster_mean)

    bpos = jnp.arange(_B, dtype=jnp.int32)
    winner = jnp.zeros((_N,), jnp.int32).at[indexes].max(bpos)
    wb = winner[indexes]
    mask = (wb == bpos).astype(jnp.float32)[:, None]

    tile_of = jnp.minimum(indexes // _MTROWS, _NTILES - 1)
    counts = jnp.bincount(tile_of, length=_NTILES).astype(jnp.int32)
    counts_bc = jnp.broadcast_to(counts[:, None], (_NTILES, 16))
    starts = (jnp.cumsum(counts) - counts).astype(jnp.int32)
    order = jnp.argsort(tile_of, stable=True).astype(jnp.int32)
    jj = jnp.arange(_B, dtype=jnp.int32)[None, :]
    pos = jnp.where(jj < counts[:, None], starts[:, None] + jj, starts[:, None])
    pos = jnp.minimum(pos, _B - 1)
    rb = order[pos]
    rsrc = wb[rb].reshape(_NTILES, _MBCH, _MCH)
    rdst = indexes[rb].reshape(_NTILES, _MBCH, _MCH)

    labels_p = jnp.concatenate(
        [labels, jnp.full((_LPAD - _N,), -1, jnp.int32)]
    ).reshape(_LCHUNKS, _SCH)
    targets_p = targets.reshape(_TCHUNKS, _SCH)
    indexes2d = indexes.reshape(_NTILES, _GCH)

    zeros_d = jnp.zeros((64, _D), jnp.float32)
    zeros16 = jnp.zeros((64, 16), jnp.float32)
    zeros_i = jnp.zeros((_SCH,), jnp.int32)

    old = _gather_old(features, indexes2d)
    new_rows, delta = _update_rows(old, inputs, mask)
    feat_out = _copy_scatter(features, new_rows, rsrc, rdst, counts_bc)
    mean, flg = _seg(features, labels_p, targets_p, delta, zeros_d, zeros16,
                     zeros_i)
    cm_out = _merge(mean, flg, cluster_mean)
    return (logits, feat_out, cm_out)


# speedup vs baseline: 4.7079x; 4.7079x over previous
import jax
import jax.numpy as jnp
from jax import lax
from jax.experimental import pallas as pl
from jax.experimental.pallas import tpu as pltpu
from jax.experimental.pallas import tpu_sc as plsc

_N = 100000
_C = 5000
_D = 256
_B = 4096
_TEMP = 0.05
_MOM = 0.2

_NTILES = 32
_GCH = 128

_MCH = 128
_MFULL = _N // _MCH
_MTAIL = _N - _MFULL * _MCH
_MTAILS = _MFULL * _MCH
_MCPT = 25
_MTROWS = _MCPT * _MCH
_MBCH = _B // _MCH

_OWN = 160
_SCH = 1024
_LCHUNKS = 98
_LPAD = _LCHUNKS * _SCH
_TCHUNKS = _B // _SCH
_CAP = 2048


def _sc_mesh():
    return plsc.VectorSubcoreMesh(core_axis_name="c", subcore_axis_name="s")


def _mm_body(a_ref, b_ref, o_ref):
    o_ref[...] = lax.dot_general(
        a_ref[...], b_ref[...], (((1,), (1,)), ((), ())),
        preferred_element_type=jnp.float32) / _TEMP


def _logits(inputs, cluster_mean):
    bm, bn = 512, 512
    grid = (_B // bm, pl.cdiv(_C, bn))
    return pl.pallas_call(
        _mm_body,
        grid=grid,
        in_specs=[pl.BlockSpec((bm, _D), lambda i, j: (i, 0)),
                  pl.BlockSpec((bn, _D), lambda i, j: (j, 0))],
        out_specs=pl.BlockSpec((bm, bn), lambda i, j: (i, j)),
        out_shape=jax.ShapeDtypeStruct((_B, _C), jnp.float32),
        compiler_params=pltpu.CompilerParams(
            dimension_semantics=("parallel", "parallel")),
    )(inputs, cluster_mean)


def _gather_body(feat, idx2d, out, idx_v, rows_v, sem):
    wid = lax.axis_index("s") * 2 + lax.axis_index("c")
    pltpu.sync_copy(idx2d.at[wid], idx_v)
    pltpu.async_copy(feat.at[idx_v], rows_v, sem).wait()
    pltpu.sync_copy(rows_v, out.at[pl.ds(wid * _GCH, _GCH)])


def _gather_old(features, indexes2d):
    fn = pl.kernel(
        _gather_body,
        out_type=jax.ShapeDtypeStruct((_B, _D), jnp.float32),
        mesh=_sc_mesh(),
        scratch_types=[pltpu.VMEM((_GCH,), jnp.int32),
                       pltpu.VMEM((_GCH, _D), jnp.float32),
                       pltpu.SemaphoreType.DMA],
    )
    return fn(features, indexes2d)


def _upd_body(old_ref, inp_ref, msk_ref, new_ref, dlt_ref):
    old = old_ref[...]
    pre = _MOM * old + (1.0 - _MOM) * inp_ref[...]
    nrm = jnp.sqrt(jnp.sum(pre * pre, axis=1, keepdims=True))
    new = pre / jnp.maximum(nrm, 1e-12)
    new_ref[...] = new
    dlt_ref[...] = (new - old) * msk_ref[...]


def _update_rows(old, inputs, mask):
    bm = 512
    grid = (_B // bm,)
    out_sds = jax.ShapeDtypeStruct((_B, _D), jnp.float32)
    return pl.pallas_call(
        _upd_body,
        grid=grid,
        in_specs=[pl.BlockSpec((bm, _D), lambda i: (i, 0)),
                  pl.BlockSpec((bm, _D), lambda i: (i, 0)),
                  pl.BlockSpec((bm, 1), lambda i: (i, 0))],
        out_specs=[pl.BlockSpec((bm, _D), lambda i: (i, 0)),
                   pl.BlockSpec((bm, _D), lambda i: (i, 0))],
        out_shape=[out_sds, out_sds],
        compiler_params=pltpu.CompilerParams(dimension_semantics=("parallel",)),
    )(old, inputs, mask)


def _copy_body(feat, new_rows, rsrc, rdst, counts, feat_out,
               rows_v, rs_v, rd_v, cnt_v, sem):
    wid = lax.axis_index("s") * 2 + lax.axis_index("c")
    pltpu.sync_copy(counts.at[wid], cnt_v)

    n_my = jnp.where(wid == _NTILES - 1,
                     _MFULL - (_NTILES - 1) * _MCPT, _MCPT)

    def body1(c, carry):
        @pl.when(c < n_my)
        def _():
            gc = wid * _MCPT + c
            pltpu.sync_copy(feat.at[pl.ds(gc * _MCH, _MCH)], rows_v)
            pltpu.sync_copy(rows_v, feat_out.at[pl.ds(gc * _MCH, _MCH)])
        return carry

    lax.fori_loop(0, _MCPT, body1, 0)

    @pl.when(wid == _NTILES - 1)
    def _tail():
        pltpu.sync_copy(feat.at[pl.ds(_MTAILS, _MTAIL)],
                        rows_v.at[pl.ds(0, _MTAIL)])
        pltpu.sync_copy(rows_v.at[pl.ds(0, _MTAIL)],
                        feat_out.at[pl.ds(_MTAILS, _MTAIL)])

    n_upd = cnt_v[...][0]
    brow = cnt_v[...][1]

    def body2(ch, carry):
        @pl.when(ch * _MCH < n_upd)
        def _():
            pltpu.sync_copy(rsrc.at[brow + ch], rs_v)
            pltpu.sync_copy(rdst.at[brow + ch], rd_v)
            pltpu.async_copy(new_rows.at[rs_v], rows_v, sem).wait()
            pltpu.sync_copy(rows_v, feat_out.at[rd_v])
        return carry

    lax.fori_loop(0, _MBCH, body2, 0)


def _copy_scatter(features, new_rows, rsrc, rdst, counts):
    fn = pl.kernel(
        _copy_body,
        out_type=jax.ShapeDtypeStruct((_N, _D), jnp.float32),
        mesh=_sc_mesh(),
        scratch_types=[pltpu.VMEM((_MCH, _D), jnp.float32),
                       pltpu.VMEM((_MCH,), jnp.int32),
                       pltpu.VMEM((_MCH,), jnp.int32),
                       pltpu.VMEM((16,), jnp.int32),
                       pltpu.SemaphoreType.DMA],
    )
    return fn(features, new_rows, rsrc, rdst, counts)


def _seg_body(feat, labels_p, targets_p, delta, zeros_d, zeros16, zeros_i,
              mean_out, flg_out,
              acc_v, cnt_v, flg_v, lbl_v, list_v, clist_v, rows_v, sem):
    wid = lax.axis_index("s") * 2 + lax.axis_index("c")
    lo = jnp.minimum(wid * _OWN, _C - _OWN)
    iota16 = lax.iota(jnp.int32, 16)
    ones16 = jnp.full((16,), 1.0, jnp.float32)

    pltpu.sync_copy(zeros_d, acc_v.at[pl.ds(0, 64)])
    pltpu.sync_copy(zeros_d, acc_v.at[pl.ds(64, 64)])
    pltpu.sync_copy(zeros_d.at[pl.ds(0, 32)], acc_v.at[pl.ds(128, 32)])
    for tbl in (cnt_v, flg_v):
        pltpu.sync_copy(zeros16, tbl.at[pl.ds(0, 64)])
        pltpu.sync_copy(zeros16, tbl.at[pl.ds(64, 64)])
        pltpu.sync_copy(zeros16.at[pl.ds(0, 32)], tbl.at[pl.ds(128, 32)])
    pltpu.sync_copy(zeros_i, list_v.at[pl.ds(0, _SCH)])
    pltpu.sync_copy(zeros_i, list_v.at[pl.ds(_SCH, _SCH)])

    def drain(src, cursor):
        def dbody(g, carry):
            @pl.when(g * 16 < cursor)
            def _():
                idxsl = list_v.at[pl.ds(g * 16, 16)]
                pltpu.async_copy(src.at[idxsl], rows_v, sem).wait()
                cvec = clist_v[pl.ds(g * 16, 16)]
                for k in range(16):
                    jk = cvec[k]

                    @pl.when(g * 16 + k < cursor)
                    def _(jk=jk, k=k):
                        for cc in range(16):
                            sl = pl.ds(cc * 16, 16)
                            acc_v[jk, sl] = acc_v[jk, sl] + rows_v[k, sl]
            return carry

        lax.fori_loop(0, _CAP // 16, dbody, 0)

    def compress_scan(tab, nch, src, match_table):
        def chunk_body(c, cursor):
            pltpu.sync_copy(tab.at[c], lbl_v)

            def grp(k, cur):
                tv = lbl_v[pl.ds(k * 16, 16)]
                rel = tv - lo
                m = (rel >= 0) & (rel < _OWN)
                relc = jnp.clip(rel, 0, _OWN - 1)
                plsc.addupdate_scatter(match_table, [relc, iota16], ones16,
                                       mask=m)
                rid = c * _SCH + k * 16 + iota16
                plsc.store_compressed(list_v.at[pl.ds(cur, 16)], rid, mask=m)
                plsc.store_compressed(clist_v.at[pl.ds(cur, 16)], relc, mask=m)
                return cur + plsc.all_reduce_population_count(m)[0]

            cursor = lax.fori_loop(0, _SCH // 16, grp, cursor)
            full = cursor > _CAP - _SCH

            @pl.when(full)
            def _():
                drain(src, cursor)

            return jnp.where(full, 0, cursor)

        cursor = lax.fori_loop(0, nch, chunk_body, 0)

        @pl.when(cursor > 0)
        def _():
            drain(src, cursor)

    compress_scan(targets_p, _TCHUNKS, delta, flg_v)
    compress_scan(labels_p, _LCHUNKS, feat, cnt_v)

    def fin(j, carry):
        cs = jnp.maximum(jnp.sum(cnt_v[j, pl.ds(0, 16)]), 1.0)
        for cc in range(16):
            sl = pl.ds(cc * 16, 16)
            acc_v[j, sl] = acc_v[j, sl] / cs
        return carry

    lax.fori_loop(0, _OWN, fin, 0)
    pltpu.sync_copy(acc_v.at[pl.ds(0, _OWN)], mean_out.at[pl.ds(lo, _OWN)])
    pltpu.sync_copy(flg_v.at[pl.ds(0, _OWN)], flg_out.at[pl.ds(lo, _OWN)])


def _seg(features, labels_p, targets_p, delta, zeros_d, zeros16, zeros_i):
    fn = pl.kernel(
        _seg_body,
        out_type=(jax.ShapeDtypeStruct((_C, _D), jnp.float32),
                  jax.ShapeDtypeStruct((_C, 16), jnp.float32)),
        mesh=_sc_mesh(),
        scratch_types=[pltpu.VMEM((160, _D), jnp.float32),
                       pltpu.VMEM((160, 16), jnp.float32),
                       pltpu.VMEM((160, 16), jnp.float32),
                       pltpu.VMEM((_SCH,), jnp.int32),
                       pltpu.VMEM((_CAP,), jnp.int32),
                       pltpu.VMEM((_CAP,), jnp.int32),
                       pltpu.VMEM((16, _D), jnp.float32),
                       pltpu.SemaphoreType.DMA],
        compiler_params=pltpu.CompilerParams(needs_layout_passes=False),
    )
    return fn(features, labels_p, targets_p, delta, zeros_d, zeros16, zeros_i)


def _merge_body(mean_ref, flg_ref, cm_ref, o_ref):
    f = jnp.sum(flg_ref[...], axis=1, keepdims=True)
    o_ref[...] = jnp.where(f > 0.5, mean_ref[...], cm_ref[...])


def _merge(mean, flg, cluster_mean):
    bn = 1000
    grid = (_C // bn,)
    wide = pl.BlockSpec((bn, _D), lambda i: (i, 0))
    narrow = pl.BlockSpec((bn, 16), lambda i: (i, 0))
    return pl.pallas_call(
        _merge_body,
        grid=grid,
        in_specs=[wide, narrow, wide],
        out_specs=wide,
        out_shape=jax.ShapeDtypeStruct((_C, _D), jnp.float32),
        compiler_params=pltpu.CompilerParams(dimension_semantics=("parallel",)),
    )(mean, flg, cluster_mean)


def kernel(inputs, indexes, IoU, targets, features, labels, cluster_mean):
    del IoU
    indexes = indexes.astype(jnp.int32)
    labels = labels.astype(jnp.int32)
    targets = targets.astype(jnp.int32)

    logits = _logits(inputs, cluster_mean)

    bpos = jnp.arange(_B, dtype=jnp.int32)
    winner = jnp.zeros((_N,), jnp.int32).at[indexes].max(bpos)
    wb = winner[indexes]
    mask = (wb == bpos).astype(jnp.float32)[:, None]

    tile_of = jnp.minimum(indexes // _MTROWS, _NTILES - 1)
    counts = jnp.bincount(tile_of, length=_NTILES).astype(jnp.int32)
    starts = (jnp.cumsum(counts) - counts).astype(jnp.int32)
    order = jnp.argsort(tile_of, stable=True).astype(jnp.int32)
    cnt_pad = ((counts + _MCH - 1) // _MCH) * _MCH
    bases = (jnp.cumsum(cnt_pad) - cnt_pad).astype(jnp.int32)
    meta = jnp.zeros((_NTILES, 16), jnp.int32)
    meta = meta.at[:, 0].set(counts).at[:, 1].set(bases // _MCH)
    rt = _B + _NTILES * _MCH
    j = jnp.arange(rt, dtype=jnp.int32)
    tslot = jnp.minimum(
        jnp.searchsorted(bases + cnt_pad, j, side="right").astype(jnp.int32),
        _NTILES - 1)
    off = jnp.minimum(j - bases[tslot], jnp.maximum(counts[tslot] - 1, 0))
    src_pos = jnp.minimum(starts[tslot] + off, _B - 1)
    rbc = order[src_pos]
    rsrc = wb[rbc].reshape(rt // _MCH, _MCH)
    rdst = indexes[rbc].reshape(rt // _MCH, _MCH)

    labels_p = jnp.concatenate(
        [labels, jnp.full((_LPAD - _N,), -1, jnp.int32)]
    ).reshape(_LCHUNKS, _SCH)
    targets_p = targets.reshape(_TCHUNKS, _SCH)
    indexes2d = indexes.reshape(_NTILES, _GCH)

    zeros_d = jnp.zeros((64, _D), jnp.float32)
    zeros16 = jnp.zeros((64, 16), jnp.float32)
    zeros_i = jnp.zeros((_SCH,), jnp.int32)

    old = _gather_old(features, indexes2d)
    new_rows, delta = _update_rows(old, inputs, mask)
    feat_out = _copy_scatter(features, new_rows, rsrc, rdst, meta)
    mean, flg = _seg(features, labels_p, targets_p, delta, zeros_d, zeros16,
                     zeros_i)
    cm_out = _merge(mean, flg, cluster_mean)
    return (logits, feat_out, cm_out)


# speedup vs baseline: 4.7554x; 1.0101x over previous
import jax
import jax.numpy as jnp
from jax import lax
from jax.experimental import pallas as pl
from jax.experimental.pallas import tpu as pltpu
from jax.experimental.pallas import tpu_sc as plsc

_N = 100000
_C = 5000
_D = 256
_B = 4096
_TEMP = 0.05
_MOM = 0.2

_NTILES = 32
_GCH = 128

_MCH = 128
_MFULL = _N // _MCH
_MTAIL = _N - _MFULL * _MCH
_MTAILS = _MFULL * _MCH
_MCPT = 25
_MTROWS = _MCPT * _MCH
_MBCH = _B // _MCH

_OWN = 160
_SCH = 1024
_LCHUNKS = 98
_LPAD = _LCHUNKS * _SCH
_TCHUNKS = _B // _SCH
_CAP = 2048


def _sc_mesh():
    return plsc.VectorSubcoreMesh(core_axis_name="c", subcore_axis_name="s")


def _mm_body(a_ref, b_ref, o_ref):
    o_ref[...] = lax.dot_general(
        a_ref[...], b_ref[...], (((1,), (1,)), ((), ())),
        preferred_element_type=jnp.float32) / _TEMP


def _logits(inputs, cluster_mean):
    bm, bn = 512, 512
    grid = (_B // bm, pl.cdiv(_C, bn))
    return pl.pallas_call(
        _mm_body,
        grid=grid,
        in_specs=[pl.BlockSpec((bm, _D), lambda i, j: (i, 0)),
                  pl.BlockSpec((bn, _D), lambda i, j: (j, 0))],
        out_specs=pl.BlockSpec((bm, bn), lambda i, j: (i, j)),
        out_shape=jax.ShapeDtypeStruct((_B, _C), jnp.float32),
        compiler_params=pltpu.CompilerParams(
            dimension_semantics=("parallel", "parallel")),
    )(inputs, cluster_mean)


def _gather_body(feat, idx2d, out, idx_v, rows_v, sem):
    wid = lax.axis_index("s") * 2 + lax.axis_index("c")
    pltpu.sync_copy(idx2d.at[wid], idx_v)
    pltpu.async_copy(feat.at[idx_v], rows_v, sem).wait()
    pltpu.sync_copy(rows_v, out.at[pl.ds(wid * _GCH, _GCH)])


def _gather_old(features, indexes2d):
    fn = pl.kernel(
        _gather_body,
        out_type=jax.ShapeDtypeStruct((_B, _D), jnp.float32),
        mesh=_sc_mesh(),
        scratch_types=[pltpu.VMEM((_GCH,), jnp.int32),
                       pltpu.VMEM((_GCH, _D), jnp.float32),
                       pltpu.SemaphoreType.DMA],
    )
    return fn(features, indexes2d)


def _upd_body(old_ref, inp_ref, msk_ref, new_ref, dlt_ref):
    old = old_ref[...]
    pre = _MOM * old + (1.0 - _MOM) * inp_ref[...]
    nrm = jnp.sqrt(jnp.sum(pre * pre, axis=1, keepdims=True))
    new = pre / jnp.maximum(nrm, 1e-12)
    new_ref[...] = new
    dlt_ref[...] = (new - old) * msk_ref[...]


def _update_rows(old, inputs, mask):
    bm = 512
    grid = (_B // bm,)
    out_sds = jax.ShapeDtypeStruct((_B, _D), jnp.float32)
    return pl.pallas_call(
        _upd_body,
        grid=grid,
        in_specs=[pl.BlockSpec((bm, _D), lambda i: (i, 0)),
                  pl.BlockSpec((bm, _D), lambda i: (i, 0)),
                  pl.BlockSpec((bm, 1), lambda i: (i, 0))],
        out_specs=[pl.BlockSpec((bm, _D), lambda i: (i, 0)),
                   pl.BlockSpec((bm, _D), lambda i: (i, 0))],
        out_shape=[out_sds, out_sds],
        compiler_params=pltpu.CompilerParams(dimension_semantics=("parallel",)),
    )(old, inputs, mask)


def _copy_body(feat, new_rows, rsrc, rdst, counts, feat_out,
               rows_v, rs_v, rd_v, cnt_v, sem):
    wid = lax.axis_index("s") * 2 + lax.axis_index("c")
    pltpu.sync_copy(counts.at[wid], cnt_v)

    n_my = jnp.where(wid == _NTILES - 1,
                     _MFULL - (_NTILES - 1) * _MCPT, _MCPT)

    def body1(c, carry):
        @pl.when(c < n_my)
        def _():
            gc = wid * _MCPT + c
            pltpu.sync_copy(feat.at[pl.ds(gc * _MCH, _MCH)], rows_v)
            pltpu.sync_copy(rows_v, feat_out.at[pl.ds(gc * _MCH, _MCH)])
        return carry

    lax.fori_loop(0, _MCPT, body1, 0)

    @pl.when(wid == _NTILES - 1)
    def _tail():
        pltpu.sync_copy(feat.at[pl.ds(_MTAILS, _MTAIL)],
                        rows_v.at[pl.ds(0, _MTAIL)])
        pltpu.sync_copy(rows_v.at[pl.ds(0, _MTAIL)],
                        feat_out.at[pl.ds(_MTAILS, _MTAIL)])

    n_upd = cnt_v[...][0]
    brow = cnt_v[...][1]

    def body2(ch, carry):
        @pl.when(ch * _MCH < n_upd)
        def _():
            pltpu.sync_copy(rsrc.at[brow + ch], rs_v)
            pltpu.sync_copy(rdst.at[brow + ch], rd_v)
            pltpu.async_copy(new_rows.at[rs_v], rows_v, sem).wait()
            pltpu.sync_copy(rows_v, feat_out.at[rd_v])
        return carry

    lax.fori_loop(0, _MBCH, body2, 0)


def _copy_scatter(features, new_rows, rsrc, rdst, counts):
    fn = pl.kernel(
        _copy_body,
        out_type=jax.ShapeDtypeStruct((_N, _D), jnp.float32),
        mesh=_sc_mesh(),
        scratch_types=[pltpu.VMEM((_MCH, _D), jnp.float32),
                       pltpu.VMEM((_MCH,), jnp.int32),
                       pltpu.VMEM((_MCH,), jnp.int32),
                       pltpu.VMEM((16,), jnp.int32),
                       pltpu.SemaphoreType.DMA],
    )
    return fn(features, new_rows, rsrc, rdst, counts)


def _seg_body(feat, labels_p, targets_p, delta, zeros_d, zeros16, zeros_i,
              mean_out, flg_out,
              acc_v, cnt_v, flg_v, lbl_v, list_v, clist_v, rows_v, sem):
    wid = lax.axis_index("s") * 2 + lax.axis_index("c")
    lo = jnp.minimum(wid * _OWN, _C - _OWN)
    iota16 = lax.iota(jnp.int32, 16)
    ones16 = jnp.full((16,), 1.0, jnp.float32)

    pltpu.sync_copy(zeros_d, acc_v.at[pl.ds(0, 64)])
    pltpu.sync_copy(zeros_d, acc_v.at[pl.ds(64, 64)])
    pltpu.sync_copy(zeros_d.at[pl.ds(0, 32)], acc_v.at[pl.ds(128, 32)])
    for tbl in (cnt_v, flg_v):
        pltpu.sync_copy(zeros16, tbl.at[pl.ds(0, 64)])
        pltpu.sync_copy(zeros16, tbl.at[pl.ds(64, 64)])
        pltpu.sync_copy(zeros16.at[pl.ds(0, 32)], tbl.at[pl.ds(128, 32)])
    pltpu.sync_copy(zeros_i, list_v.at[pl.ds(0, _SCH)])
    pltpu.sync_copy(zeros_i, list_v.at[pl.ds(_SCH, _SCH)])

    def drain(src, cursor):
        def dbody(g, carry):
            @pl.when(g * 16 < cursor)
            def _():
                idxsl = list_v.at[pl.ds(g * 16, 16)]
                pltpu.async_copy(src.at[idxsl], rows_v, sem).wait()
                cvec = clist_v[pl.ds(g * 16, 16)]
                for k in range(16):
                    jk = cvec[k]

                    @pl.when(g * 16 + k < cursor)
                    def _(jk=jk, k=k):
                        for cc in range(16):
                            sl = pl.ds(cc * 16, 16)
                            acc_v[jk, sl] = acc_v[jk, sl] + rows_v[k, sl]
            return carry

        lax.fori_loop(0, _CAP // 16, dbody, 0)

    def compress_scan(tab, nch, src, match_table, filt=False):
        def chunk_body(c, cursor):
            pltpu.sync_copy(tab.at[c], lbl_v)

            def grp(k, cur):
                tv = lbl_v[pl.ds(k * 16, 16)]
                rel = tv - lo
                m0 = (rel >= 0) & (rel < _OWN)
                relc = jnp.clip(rel, 0, _OWN - 1)
                plsc.addupdate_scatter(match_table, [relc, iota16], ones16,
                                       mask=m0)
                if filt:
                    fl = plsc.load_gather(flg_v, [relc, iota16])
                    m = m0 & (fl > 0.5)
                else:
                    m = m0
                rid = c * _SCH + k * 16 + iota16
                plsc.store_compressed(list_v.at[pl.ds(cur, 16)], rid, mask=m)
                plsc.store_compressed(clist_v.at[pl.ds(cur, 16)], relc, mask=m)
                return cur + plsc.all_reduce_population_count(m)[0]

            cursor = lax.fori_loop(0, _SCH // 16, grp, cursor)
            full = cursor > _CAP - _SCH

            @pl.when(full)
            def _():
                drain(src, cursor)

            return jnp.where(full, 0, cursor)

        cursor = lax.fori_loop(0, nch, chunk_body, 0)

        @pl.when(cursor > 0)
        def _():
            drain(src, cursor)

    compress_scan(targets_p, _TCHUNKS, delta, flg_v)

    def collapse(j, carry):
        s = jnp.sum(flg_v[j, pl.ds(0, 16)])
        flg_v[j, pl.ds(0, 16)] = jnp.zeros((16,), jnp.float32) + s
        return carry

    lax.fori_loop(0, _OWN, collapse, 0)
    compress_scan(labels_p, _LCHUNKS, feat, cnt_v, filt=True)

    def fin(j, carry):
        cs = jnp.maximum(jnp.sum(cnt_v[j, pl.ds(0, 16)]), 1.0)
        for cc in range(16):
            sl = pl.ds(cc * 16, 16)
            acc_v[j, sl] = acc_v[j, sl] / cs
        return carry

    lax.fori_loop(0, _OWN, fin, 0)
    pltpu.sync_copy(acc_v.at[pl.ds(0, _OWN)], mean_out.at[pl.ds(lo, _OWN)])
    pltpu.sync_copy(flg_v.at[pl.ds(0, _OWN)], flg_out.at[pl.ds(lo, _OWN)])


def _seg(features, labels_p, targets_p, delta, zeros_d, zeros16, zeros_i):
    fn = pl.kernel(
        _seg_body,
        out_type=(jax.ShapeDtypeStruct((_C, _D), jnp.float32),
                  jax.ShapeDtypeStruct((_C, 16), jnp.float32)),
        mesh=_sc_mesh(),
        scratch_types=[pltpu.VMEM((160, _D), jnp.float32),
                       pltpu.VMEM((160, 16), jnp.float32),
                       pltpu.VMEM((160, 16), jnp.float32),
                       pltpu.VMEM((_SCH,), jnp.int32),
                       pltpu.VMEM((_CAP,), jnp.int32),
                       pltpu.VMEM((_CAP,), jnp.int32),
                       pltpu.VMEM((16, _D), jnp.float32),
                       pltpu.SemaphoreType.DMA],
        compiler_params=pltpu.CompilerParams(needs_layout_passes=False),
    )
    return fn(features, labels_p, targets_p, delta, zeros_d, zeros16, zeros_i)


def _merge_body(mean_ref, flg_ref, cm_ref, o_ref):
    f = jnp.sum(flg_ref[...], axis=1, keepdims=True)
    o_ref[...] = jnp.where(f > 0.5, mean_ref[...], cm_ref[...])


def _merge(mean, flg, cluster_mean):
    bn = 1000
    grid = (_C // bn,)
    wide = pl.BlockSpec((bn, _D), lambda i: (i, 0))
    narrow = pl.BlockSpec((bn, 16), lambda i: (i, 0))
    return pl.pallas_call(
        _merge_body,
        grid=grid,
        in_specs=[wide, narrow, wide],
        out_specs=wide,
        out_shape=jax.ShapeDtypeStruct((_C, _D), jnp.float32),
        compiler_params=pltpu.CompilerParams(dimension_semantics=("parallel",)),
    )(mean, flg, cluster_mean)


def kernel(inputs, indexes, IoU, targets, features, labels, cluster_mean):
    del IoU
    indexes = indexes.astype(jnp.int32)
    labels = labels.astype(jnp.int32)
    targets = targets.astype(jnp.int32)

    logits = _logits(inputs, cluster_mean)

    bpos = jnp.arange(_B, dtype=jnp.int32)
    winner = jnp.zeros((_N,), jnp.int32).at[indexes].max(bpos)
    wb = winner[indexes]
    mask = (wb == bpos).astype(jnp.float32)[:, None]

    tile_of = jnp.minimum(indexes // _MTROWS, _NTILES - 1)
    counts = jnp.bincount(tile_of, length=_NTILES).astype(jnp.int32)
    starts = (jnp.cumsum(counts) - counts).astype(jnp.int32)
    order = jnp.argsort(tile_of, stable=True).astype(jnp.int32)
    cnt_pad = ((counts + _MCH - 1) // _MCH) * _MCH
    bases = (jnp.cumsum(cnt_pad) - cnt_pad).astype(jnp.int32)
    meta = jnp.zeros((_NTILES, 16), jnp.int32)
    meta = meta.at[:, 0].set(counts).at[:, 1].set(bases // _MCH)
    rt = _B + _NTILES * _MCH
    j = jnp.arange(rt, dtype=jnp.int32)
    tslot = jnp.minimum(
        jnp.searchsorted(bases + cnt_pad, j, side="right").astype(jnp.int32),
        _NTILES - 1)
    off = jnp.minimum(j - bases[tslot], jnp.maximum(counts[tslot] - 1, 0))
    src_pos = jnp.minimum(starts[tslot] + off, _B - 1)
    rbc = order[src_pos]
    rsrc = wb[rbc].reshape(rt // _MCH, _MCH)
    rdst = indexes[rbc].reshape(rt // _MCH, _MCH)

    labels_p = jnp.concatenate(
        [labels, jnp.full((_LPAD - _N,), -1, jnp.int32)]
    ).reshape(_LCHUNKS, _SCH)
    targets_p = targets.reshape(_TCHUNKS, _SCH)
    indexes2d = indexes.reshape(_NTILES, _GCH)

    zeros_d = jnp.zeros((64, _D), jnp.float32)
    zeros16 = jnp.zeros((64, 16), jnp.float32)
    zeros_i = jnp.zeros((_SCH,), jnp.int32)

    old = _gather_old(features, indexes2d)
    new_rows, delta = _update_rows(old, inputs, mask)
    feat_out = _copy_scatter(features, new_rows, rsrc, rdst, meta)
    mean, flg = _seg(features, labels_p, targets_p, delta, zeros_d, zeros16,
                     zeros_i)
    cm_out = _merge(mean, flg, cluster_mean)
    return (logits, feat_out, cm_out)


# speedup vs baseline: 4.7582x; 1.0006x over previous
import jax
import jax.numpy as jnp
from jax import lax
from jax.experimental import pallas as pl
from jax.experimental.pallas import tpu as pltpu
from jax.experimental.pallas import tpu_sc as plsc

_N = 100000
_C = 5000
_D = 256
_B = 4096
_TEMP = 0.05
_MOM = 0.2

_NTILES = 32
_GCH = 128

_MCH = 128
_MFULL = _N // _MCH
_MTAIL = _N - _MFULL * _MCH
_MTAILS = _MFULL * _MCH
_MCPT = 25
_MTROWS = _MCPT * _MCH
_MBCH = _B // _MCH

_OWN = 160
_SCH = 1024
_LCHUNKS = 98
_LPAD = _LCHUNKS * _SCH
_TCHUNKS = _B // _SCH
_CAP = 2048


def _sc_mesh():
    return plsc.VectorSubcoreMesh(core_axis_name="c", subcore_axis_name="s")


def _mm_body(a_ref, b_ref, o_ref):
    o_ref[...] = lax.dot_general(
        a_ref[...], b_ref[...], (((1,), (1,)), ((), ())),
        preferred_element_type=jnp.float32) / _TEMP


def _logits(inputs, cluster_mean):
    bm, bn = 512, 512
    grid = (_B // bm, pl.cdiv(_C, bn))
    return pl.pallas_call(
        _mm_body,
        grid=grid,
        in_specs=[pl.BlockSpec((bm, _D), lambda i, j: (i, 0)),
                  pl.BlockSpec((bn, _D), lambda i, j: (j, 0))],
        out_specs=pl.BlockSpec((bm, bn), lambda i, j: (i, j)),
        out_shape=jax.ShapeDtypeStruct((_B, _C), jnp.float32),
        compiler_params=pltpu.CompilerParams(
            dimension_semantics=("parallel", "parallel")),
    )(inputs, cluster_mean)


def _gather_body(feat, idx2d, out, idx_v, rows_v, sem):
    wid = lax.axis_index("s") * 2 + lax.axis_index("c")
    pltpu.sync_copy(idx2d.at[wid], idx_v)
    pltpu.async_copy(feat.at[idx_v], rows_v, sem).wait()
    pltpu.sync_copy(rows_v, out.at[pl.ds(wid * _GCH, _GCH)])


def _gather_old(features, indexes2d):
    fn = pl.kernel(
        _gather_body,
        out_type=jax.ShapeDtypeStruct((_B, _D), jnp.float32),
        mesh=_sc_mesh(),
        scratch_types=[pltpu.VMEM((_GCH,), jnp.int32),
                       pltpu.VMEM((_GCH, _D), jnp.float32),
                       pltpu.SemaphoreType.DMA],
    )
    return fn(features, indexes2d)


def _upd_body(old_ref, inp_ref, msk_ref, new_ref, dlt_ref):
    old = old_ref[...]
    pre = _MOM * old + (1.0 - _MOM) * inp_ref[...]
    nrm = jnp.sqrt(jnp.sum(pre * pre, axis=1, keepdims=True))
    new = pre / jnp.maximum(nrm, 1e-12)
    new_ref[...] = new
    dlt_ref[...] = (new - old) * msk_ref[...]


def _update_rows(old, inputs, mask):
    bm = 512
    grid = (_B // bm,)
    out_sds = jax.ShapeDtypeStruct((_B, _D), jnp.float32)
    return pl.pallas_call(
        _upd_body,
        grid=grid,
        in_specs=[pl.BlockSpec((bm, _D), lambda i: (i, 0)),
                  pl.BlockSpec((bm, _D), lambda i: (i, 0)),
                  pl.BlockSpec((bm, 1), lambda i: (i, 0))],
        out_specs=[pl.BlockSpec((bm, _D), lambda i: (i, 0)),
                   pl.BlockSpec((bm, _D), lambda i: (i, 0))],
        out_shape=[out_sds, out_sds],
        compiler_params=pltpu.CompilerParams(dimension_semantics=("parallel",)),
    )(old, inputs, mask)


def _copy_body(feat, new_rows, rsrc, rdst, counts, feat_out,
               rows_v, rs_v, rd_v, cnt_v, sem):
    wid = lax.axis_index("s") * 2 + lax.axis_index("c")
    pltpu.sync_copy(counts.at[wid], cnt_v)

    n_my = jnp.where(wid == _NTILES - 1,
                     _MFULL - (_NTILES - 1) * _MCPT, _MCPT)

    def body1(c, carry):
        @pl.when(c < n_my)
        def _():
            gc = wid * _MCPT + c
            pltpu.sync_copy(feat.at[pl.ds(gc * _MCH, _MCH)], rows_v)
            pltpu.sync_copy(rows_v, feat_out.at[pl.ds(gc * _MCH, _MCH)])
        return carry

    lax.fori_loop(0, _MCPT, body1, 0)

    @pl.when(wid == _NTILES - 1)
    def _tail():
        pltpu.sync_copy(feat.at[pl.ds(_MTAILS, _MTAIL)],
                        rows_v.at[pl.ds(0, _MTAIL)])
        pltpu.sync_copy(rows_v.at[pl.ds(0, _MTAIL)],
                        feat_out.at[pl.ds(_MTAILS, _MTAIL)])

    n_upd = cnt_v[...][0]
    brow = cnt_v[...][1]

    def body2(ch, carry):
        @pl.when(ch * _MCH < n_upd)
        def _():
            pltpu.sync_copy(rsrc.at[brow + ch], rs_v)
            pltpu.sync_copy(rdst.at[brow + ch], rd_v)
            pltpu.async_copy(new_rows.at[rs_v], rows_v, sem).wait()
            pltpu.sync_copy(rows_v, feat_out.at[rd_v])
        return carry

    lax.fori_loop(0, _MBCH, body2, 0)


def _copy_scatter(features, new_rows, rsrc, rdst, counts):
    fn = pl.kernel(
        _copy_body,
        out_type=jax.ShapeDtypeStruct((_N, _D), jnp.float32),
        mesh=_sc_mesh(),
        scratch_types=[pltpu.VMEM((_MCH, _D), jnp.float32),
                       pltpu.VMEM((_MCH,), jnp.int32),
                       pltpu.VMEM((_MCH,), jnp.int32),
                       pltpu.VMEM((16,), jnp.int32),
                       pltpu.SemaphoreType.DMA],
    )
    return fn(features, new_rows, rsrc, rdst, counts)


def _seg_body(feat, labels_p, targets_p, delta, zeros_d, zeros16, zeros_i,
              mean_out, flg_out,
              acc_v, cnt_v, flg_v, lbl_v, list_v, clist_v, rows_v, sem):
    wid = lax.axis_index("s") * 2 + lax.axis_index("c")
    lo = jnp.minimum(wid * _OWN, _C - _OWN)
    iota16 = lax.iota(jnp.int32, 16)
    ones16 = jnp.full((16,), 1.0, jnp.float32)

    pltpu.sync_copy(zeros_d, acc_v.at[pl.ds(0, 64)])
    pltpu.sync_copy(zeros_d, acc_v.at[pl.ds(64, 64)])
    pltpu.sync_copy(zeros_d.at[pl.ds(0, 32)], acc_v.at[pl.ds(128, 32)])
    for tbl in (cnt_v, flg_v):
        pltpu.sync_copy(zeros16, tbl.at[pl.ds(0, 64)])
        pltpu.sync_copy(zeros16, tbl.at[pl.ds(64, 64)])
        pltpu.sync_copy(zeros16.at[pl.ds(0, 32)], tbl.at[pl.ds(128, 32)])
    pltpu.sync_copy(zeros_i, list_v.at[pl.ds(0, _SCH)])
    pltpu.sync_copy(zeros_i, list_v.at[pl.ds(_SCH, _SCH)])

    def drain(src, cursor):
        def dbody(g, carry):
            @pl.when(g * 32 < cursor)
            def _():
                idxsl = list_v.at[pl.ds(g * 32, 32)]
                pltpu.async_copy(src.at[idxsl], rows_v, sem).wait()
                for sub in range(2):
                    cvec = clist_v[pl.ds(g * 32 + sub * 16, 16)]
                    for k in range(16):
                        jk = cvec[k]

                        @pl.when(g * 32 + sub * 16 + k < cursor)
                        def _(jk=jk, k=k, sub=sub):
                            for cc in range(16):
                                sl = pl.ds(cc * 16, 16)
                                acc_v[jk, sl] = (acc_v[jk, sl]
                                                 + rows_v[sub * 16 + k, sl])
            return carry

        lax.fori_loop(0, _CAP // 32, dbody, 0)

    def compress_scan(tab, nch, src, match_table, filt=False):
        def chunk_body(c, cursor):
            pltpu.sync_copy(tab.at[c], lbl_v)

            def grp(k, cur):
                tv = lbl_v[pl.ds(k * 16, 16)]
                rel = tv - lo
                m0 = (rel >= 0) & (rel < _OWN)
                relc = jnp.clip(rel, 0, _OWN - 1)
                plsc.addupdate_scatter(match_table, [relc, iota16], ones16,
                                       mask=m0)
                if filt:
                    fl = plsc.load_gather(flg_v, [relc, iota16])
                    m = m0 & (fl > 0.5)
                else:
                    m = m0
                rid = c * _SCH + k * 16 + iota16
                plsc.store_compressed(list_v.at[pl.ds(cur, 16)], rid, mask=m)
                plsc.store_compressed(clist_v.at[pl.ds(cur, 16)], relc, mask=m)
                return cur + plsc.all_reduce_population_count(m)[0]

            cursor = lax.fori_loop(0, _SCH // 16, grp, cursor)
            full = cursor > _CAP - _SCH

            @pl.when(full)
            def _():
                drain(src, cursor)

            return jnp.where(full, 0, cursor)

        cursor = lax.fori_loop(0, nch, chunk_body, 0)

        @pl.when(cursor > 0)
        def _():
            drain(src, cursor)

    compress_scan(targets_p, _TCHUNKS, delta, flg_v)

    def collapse(j, carry):
        s = jnp.sum(flg_v[j, pl.ds(0, 16)])
        flg_v[j, pl.ds(0, 16)] = jnp.zeros((16,), jnp.float32) + s
        return carry

    lax.fori_loop(0, _OWN, collapse, 0)
    compress_scan(labels_p, _LCHUNKS, feat, cnt_v, filt=True)

    def fin(j, carry):
        cs = jnp.maximum(jnp.sum(cnt_v[j, pl.ds(0, 16)]), 1.0)
        for cc in range(16):
            sl = pl.ds(cc * 16, 16)
            acc_v[j, sl] = acc_v[j, sl] / cs
        return carry

    lax.fori_loop(0, _OWN, fin, 0)
    pltpu.sync_copy(acc_v.at[pl.ds(0, _OWN)], mean_out.at[pl.ds(lo, _OWN)])
    pltpu.sync_copy(flg_v.at[pl.ds(0, _OWN)], flg_out.at[pl.ds(lo, _OWN)])


def _seg(features, labels_p, targets_p, delta, zeros_d, zeros16, zeros_i):
    fn = pl.kernel(
        _seg_body,
        out_type=(jax.ShapeDtypeStruct((_C, _D), jnp.float32),
                  jax.ShapeDtypeStruct((_C, 16), jnp.float32)),
        mesh=_sc_mesh(),
        scratch_types=[pltpu.VMEM((160, _D), jnp.float32),
                       pltpu.VMEM((160, 16), jnp.float32),
                       pltpu.VMEM((160, 16), jnp.float32),
                       pltpu.VMEM((_SCH,), jnp.int32),
                       pltpu.VMEM((_CAP,), jnp.int32),
                       pltpu.VMEM((_CAP,), jnp.int32),
                       pltpu.VMEM((32, _D), jnp.float32),
                       pltpu.SemaphoreType.DMA],
        compiler_params=pltpu.CompilerParams(needs_layout_passes=False),
    )
    return fn(features, labels_p, targets_p, delta, zeros_d, zeros16, zeros_i)


def _merge_body(mean_ref, flg_ref, cm_ref, o_ref):
    f = jnp.sum(flg_ref[...], axis=1, keepdims=True)
    o_ref[...] = jnp.where(f > 0.5, mean_ref[...], cm_ref[...])


def _merge(mean, flg, cluster_mean):
    bn = 1000
    grid = (_C // bn,)
    wide = pl.BlockSpec((bn, _D), lambda i: (i, 0))
    narrow = pl.BlockSpec((bn, 16), lambda i: (i, 0))
    return pl.pallas_call(
        _merge_body,
        grid=grid,
        in_specs=[wide, narrow, wide],
        out_specs=wide,
        out_shape=jax.ShapeDtypeStruct((_C, _D), jnp.float32),
        compiler_params=pltpu.CompilerParams(dimension_semantics=("parallel",)),
    )(mean, flg, cluster_mean)


def kernel(inputs, indexes, IoU, targets, features, labels, cluster_mean):
    del IoU
    indexes = indexes.astype(jnp.int32)
    labels = labels.astype(jnp.int32)
    targets = targets.astype(jnp.int32)

    logits = _logits(inputs, cluster_mean)

    bpos = jnp.arange(_B, dtype=jnp.int32)
    winner = jnp.zeros((_N,), jnp.int32).at[indexes].max(bpos)
    wb = winner[indexes]
    mask = (wb == bpos).astype(jnp.float32)[:, None]

    tile_of = jnp.minimum(indexes // _MTROWS, _NTILES - 1)
    counts = jnp.bincount(tile_of, length=_NTILES).astype(jnp.int32)
    starts = (jnp.cumsum(counts) - counts).astype(jnp.int32)
    order = jnp.argsort(tile_of, stable=True).astype(jnp.int32)
    cnt_pad = ((counts + _MCH - 1) // _MCH) * _MCH
    bases = (jnp.cumsum(cnt_pad) - cnt_pad).astype(jnp.int32)
    meta = jnp.zeros((_NTILES, 16), jnp.int32)
    meta = meta.at[:, 0].set(counts).at[:, 1].set(bases // _MCH)
    rt = _B + _NTILES * _MCH
    j = jnp.arange(rt, dtype=jnp.int32)
    tslot = jnp.minimum(
        jnp.searchsorted(bases + cnt_pad, j, side="right").astype(jnp.int32),
        _NTILES - 1)
    off = jnp.minimum(j - bases[tslot], jnp.maximum(counts[tslot] - 1, 0))
    src_pos = jnp.minimum(starts[tslot] + off, _B - 1)
    rbc = order[src_pos]
    rsrc = wb[rbc].reshape(rt // _MCH, _MCH)
    rdst = indexes[rbc].reshape(rt // _MCH, _MCH)

    labels_p = jnp.concatenate(
        [labels, jnp.full((_LPAD - _N,), -1, jnp.int32)]
    ).reshape(_LCHUNKS, _SCH)
    targets_p = targets.reshape(_TCHUNKS, _SCH)
    indexes2d = indexes.reshape(_NTILES, _GCH)

    zeros_d = jnp.zeros((64, _D), jnp.float32)
    zeros16 = jnp.zeros((64, 16), jnp.float32)
    zeros_i = jnp.zeros((_SCH,), jnp.int32)

    old = _gather_old(features, indexes2d)
    new_rows, delta = _update_rows(old, inputs, mask)
    feat_out = _copy_scatter(features, new_rows, rsrc, rdst, meta)
    mean, flg = _seg(features, labels_p, targets_p, delta, zeros_d, zeros16,
                     zeros_i)
    cm_out = _merge(mean, flg, cluster_mean)
    return (logits, feat_out, cm_out)
